# baseline (speedup 1.0000x reference)
import functools

import jax
import jax.numpy as jnp
from jax import lax
from jax.experimental import pallas as pl
from jax.experimental.pallas import tpu as pltpu
from jax.experimental.pallas import tpu_sc as plsc

N = 50000
D = 64
E = 800000
NG = 4
GW = D // NG
NC, NS, L = 2, 16, 16

CH = 128
CPT = 400
GRP = 40
NBUF = 8
ZB = 32
NCHUNK = NS * CPT
E_PAD = NCHUNK * CH

NR = 51200
DUMMY = NR
ZR = NR // NS
GPT = N // NS
DEG_CPT = NCHUNK // (NC * NS)

BLK = 2048
NBLK = -(-N // BLK)

_mesh = plsc.VectorSubcoreMesh(
    core_axis_name="c", subcore_axis_name="s", num_cores=NC, num_subcores=NS
)
_sc_params = pltpu.CompilerParams(use_tc_tiling_on_sc=False)


@functools.partial(
    pl.kernel,
    out_type=jax.ShapeDtypeStruct((NC, NR), jnp.float32),
    mesh=_mesh,
    scratch_types=[
        pltpu.VMEM((DEG_CPT, CH), jnp.int32),
        pltpu.VMEM((CH,), jnp.float32),
        pltpu.VMEM((ZR,), jnp.float32),
        pltpu.VMEM_SHARED((NR + 128,), jnp.float32),
    ],
    compiler_params=_sc_params,
)
def _deg_kernel(dst_hbm, d_hbm, dstbuf, ones, zbuf, dacc):
    c = lax.axis_index("c")
    t = lax.axis_index("s")
    one16 = jnp.ones((L,), jnp.float32)
    zero16 = jnp.zeros((L,), jnp.float32)

    def fill1(k, _):
        ones[pl.ds(k * L, L)] = one16
        return 0

    lax.fori_loop(0, CH // L, fill1, 0)

    def fill0(k, _):
        zbuf[pl.ds(k * L, L)] = zero16
        return 0

    lax.fori_loop(0, ZR // L, fill0, 0)
    pltpu.sync_copy(zbuf, dacc.at[pl.ds(t * ZR, ZR)])
    plsc.subcore_barrier()

    base = (c * NS + t) * DEG_CPT
    pltpu.sync_copy(dst_hbm.at[pl.ds(base, DEG_CPT)], dstbuf)

    def chunk(j, _):
        pltpu.sync_copy(ones, dacc.at[dstbuf.at[j]], add=True)
        return 0

    lax.fori_loop(0, DEG_CPT, chunk, 0)
    plsc.subcore_barrier()
    pltpu.sync_copy(dacc.at[pl.ds(t * ZR, ZR)], d_hbm.at[c].at[pl.ds(t * ZR, ZR)])


@functools.partial(
    pl.kernel,
    out_type=jax.ShapeDtypeStruct((NG, NR, GW), jnp.float32),
    mesh=_mesh,
    scratch_types=[
        pltpu.VMEM((GRP, CH), jnp.int32),
        pltpu.VMEM((GRP, CH), jnp.int32),
        [pltpu.VMEM((CH, GW), jnp.float32) for _ in range(NBUF)],
        pltpu.VMEM((ZB, GW), jnp.float32),
        pltpu.VMEM_SHARED((N, GW), jnp.float32),
        pltpu.VMEM_SHARED((NR + 128, GW), jnp.float32),
        [pltpu.SemaphoreType.DMA for _ in range(NBUF)],
        [pltpu.SemaphoreType.DMA for _ in range(NBUF)],
    ],
    compiler_params=_sc_params,
)
def _edge_kernel(src_hbm, dst_hbm, g_hbm, s_hbm, srcbuf, dstbuf, rows, zbuf,
                 gbuf, acc, gsem, ssem):
    c = lax.axis_index("c")
    t = lax.axis_index("s")
    zero16 = jnp.zeros((L,), jnp.float32)

    def zfill(r, _):
        zbuf[r, pl.ds(0, L)] = zero16
        return 0

    lax.fori_loop(0, ZB, zfill, 0)

    def fire_gather(ch, b):
        pltpu.async_copy(gbuf.at[srcbuf.at[ch]], rows[b], gsem[b])

    def wait_gather(ch, b):
        pltpu.make_async_copy(gbuf.at[srcbuf.at[ch]], rows[b], gsem[b]).wait()

    def fire_scatter(ch, b):
        pltpu.async_copy(rows[b], acc.at[dstbuf.at[ch]], ssem[b], add=True)

    def wait_scatter(ch, b):
        pltpu.make_async_copy(rows[b], acc.at[dstbuf.at[ch]], ssem[b]).wait()

    for p in range(2):
        q = c * 2 + p

        def zacc(j, _):
            pltpu.sync_copy(zbuf, acc.at[pl.ds(t * ZR + j * ZB, ZB)])
            return 0

        lax.fori_loop(0, ZR // ZB, zacc, 0)
        pltpu.sync_copy(g_hbm.at[q].at[pl.ds(t * GPT, GPT)],
                        gbuf.at[pl.ds(t * GPT, GPT)])
        plsc.subcore_barrier()

        for h in range(CPT // GRP):
            base = t * CPT + h * GRP
            pltpu.sync_copy(src_hbm.at[pl.ds(base, GRP)], srcbuf)
            pltpu.sync_copy(dst_hbm.at[pl.ds(base, GRP)], dstbuf)
            for l in range(NBUF - 1):
                fire_gather(l, l)

            def step(k, _):
                for l in range(NBUF):
                    s = NBUF * k + l
                    wait_gather(s, l)
                    fire_scatter(s, l)
                    tl = (l + NBUF - 1) % NBUF

                    @pl.when(s + NBUF - 1 < GRP)
                    def _():
                        @pl.when(s >= 1)
                        def _():
                            wait_scatter(s - 1, tl)

                        fire_gather(s + NBUF - 1, tl)

                return 0

            lax.fori_loop(0, GRP // NBUF, step, 0)
            for l in range(NBUF):
                wait_scatter(GRP - NBUF + l, l)

        plsc.subcore_barrier()
        pltpu.sync_copy(acc.at[pl.ds(t * ZR, ZR)],
                        s_hbm.at[q].at[pl.ds(t * ZR, ZR)])
        plsc.subcore_barrier()


def _tc1_body(x_ref, w_ref, d_ref, g_ref, dis_ref):
    deg = d_ref[0] + d_ref[1] + 1.0
    dis = lax.rsqrt(deg)
    h = jnp.dot(x_ref[...], w_ref[...], preferred_element_type=jnp.float32)
    g = dis[:, None] * h
    for q in range(NG):
        g_ref[q] = g[:, q * GW:(q + 1) * GW]
    dis_ref[...] = dis


_tc1 = pl.pallas_call(
    _tc1_body,
    grid=(NBLK,),
    in_specs=[
        pl.BlockSpec((BLK, D), lambda i: (i, 0)),
        pl.BlockSpec((D, D), lambda i: (0, 0)),
        pl.BlockSpec((NC, BLK), lambda i: (0, i)),
    ],
    out_specs=[
        pl.BlockSpec((NG, BLK, GW), lambda i: (0, i, 0)),
        pl.BlockSpec((BLK,), lambda i: (i,)),
    ],
    out_shape=[
        jax.ShapeDtypeStruct((NG, N, GW), jnp.float32),
        jax.ShapeDtypeStruct((N,), jnp.float32),
    ],
)


def _tc2_body(s_ref, g_ref, dis_ref, b_ref, o_ref):
    dis = dis_ref[...][:, None]
    for q in range(NG):
        o_ref[:, q * GW:(q + 1) * GW] = (
            dis * (s_ref[q] + g_ref[q]) + b_ref[0, q * GW:(q + 1) * GW])


_tc2 = pl.pallas_call(
    _tc2_body,
    grid=(NBLK,),
    in_specs=[
        pl.BlockSpec((NG, BLK, GW), lambda i: (0, i, 0)),
        pl.BlockSpec((NG, BLK, GW), lambda i: (0, i, 0)),
        pl.BlockSpec((BLK,), lambda i: (i,)),
        pl.BlockSpec((1, D), lambda i: (0, 0)),
    ],
    out_specs=pl.BlockSpec((BLK, D), lambda i: (i, 0)),
    out_shape=jax.ShapeDtypeStruct((N, D), jnp.float32),
)


def kernel(x, W, b, edge_index):
    if edge_index.dtype == jnp.int64:
        ei = jax.lax.bitcast_convert_type(edge_index, jnp.int32)[:, :, 0]
    else:
        ei = edge_index.astype(jnp.int32)
    pad = E_PAD - E
    pad_dst = DUMMY + (jnp.arange(pad, dtype=jnp.int32) % 128)
    src = jnp.concatenate([ei[0], jnp.zeros((pad,), jnp.int32)]).reshape(NCHUNK, CH)
    dst = jnp.concatenate([ei[1], pad_dst]).reshape(NCHUNK, CH)

    d = _deg_kernel(dst)
    g, dis = _tc1(x, W, d)
    s = _edge_kernel(src, dst, g)
    return _tc2(s, g, dis, b.reshape(1, D))


# speedup vs baseline: 1.0086x; 1.0086x over previous
import functools

import jax
import jax.numpy as jnp
from jax import lax
from jax.experimental import pallas as pl
from jax.experimental.pallas import tpu as pltpu
from jax.experimental.pallas import tpu_sc as plsc

N = 50000
D = 64
E = 800000
NG = 4
GW = D // NG
NC, NS, L = 2, 16, 16

CH = 128
CPT = 400
GRP = 40
NBUF = 4
ZB = 32
NCHUNK = NS * CPT
E_PAD = NCHUNK * CH

NR = 51200
DUMMY = NR
ZR = NR // NS
GPT = N // NS
DEG_CPT = NCHUNK // (NC * NS)

BLK = 2048
NBLK = -(-N // BLK)

_mesh = plsc.VectorSubcoreMesh(
    core_axis_name="c", subcore_axis_name="s", num_cores=NC, num_subcores=NS
)
_sc_params = pltpu.CompilerParams(use_tc_tiling_on_sc=False)


@functools.partial(
    pl.kernel,
    out_type=jax.ShapeDtypeStruct((NC, NR), jnp.float32),
    mesh=_mesh,
    scratch_types=[
        pltpu.VMEM((DEG_CPT, CH), jnp.int32),
        pltpu.VMEM((CH,), jnp.float32),
        pltpu.VMEM((ZR,), jnp.float32),
        pltpu.VMEM_SHARED((NR + 128,), jnp.float32),
    ],
    compiler_params=_sc_params,
)
def _deg_kernel(dst_hbm, d_hbm, dstbuf, ones, zbuf, dacc):
    c = lax.axis_index("c")
    t = lax.axis_index("s")
    one16 = jnp.ones((L,), jnp.float32)
    zero16 = jnp.zeros((L,), jnp.float32)

    def fill1(k, _):
        ones[pl.ds(k * L, L)] = one16
        return 0

    lax.fori_loop(0, CH // L, fill1, 0)

    def fill0(k, _):
        zbuf[pl.ds(k * L, L)] = zero16
        return 0

    lax.fori_loop(0, ZR // L, fill0, 0)
    pltpu.sync_copy(zbuf, dacc.at[pl.ds(t * ZR, ZR)])
    plsc.subcore_barrier()

    base = (c * NS + t) * DEG_CPT
    pltpu.sync_copy(dst_hbm.at[pl.ds(base, DEG_CPT)], dstbuf)

    def chunk(j, _):
        pltpu.sync_copy(ones, dacc.at[dstbuf.at[j]], add=True)
        return 0

    lax.fori_loop(0, DEG_CPT, chunk, 0)
    plsc.subcore_barrier()
    pltpu.sync_copy(dacc.at[pl.ds(t * ZR, ZR)], d_hbm.at[c].at[pl.ds(t * ZR, ZR)])


@functools.partial(
    pl.kernel,
    out_type=jax.ShapeDtypeStruct((NG, NR, GW), jnp.float32),
    mesh=_mesh,
    scratch_types=[
        pltpu.VMEM((GRP, CH), jnp.int32),
        pltpu.VMEM((GRP, CH), jnp.int32),
        [pltpu.VMEM((CH, GW), jnp.float32) for _ in range(NBUF)],
        pltpu.VMEM((ZB, GW), jnp.float32),
        pltpu.VMEM_SHARED((N, GW), jnp.float32),
        pltpu.VMEM_SHARED((NR + 128, GW), jnp.float32),
        [pltpu.SemaphoreType.DMA for _ in range(NBUF)],
        [pltpu.SemaphoreType.DMA for _ in range(NBUF)],
    ],
    compiler_params=_sc_params,
)
def _edge_kernel(src_hbm, dst_hbm, g_hbm, s_hbm, srcbuf, dstbuf, rows, zbuf,
                 gbuf, acc, gsem, ssem):
    c = lax.axis_index("c")
    t = lax.axis_index("s")
    zero16 = jnp.zeros((L,), jnp.float32)

    def zfill(r, _):
        zbuf[r, pl.ds(0, L)] = zero16
        return 0

    lax.fori_loop(0, ZB, zfill, 0)

    def fire_gather(ch, b):
        pltpu.async_copy(gbuf.at[srcbuf.at[ch]], rows[b], gsem[b])

    def wait_gather(ch, b):
        pltpu.make_async_copy(gbuf.at[srcbuf.at[ch]], rows[b], gsem[b]).wait()

    def fire_scatter(ch, b):
        pltpu.async_copy(rows[b], acc.at[dstbuf.at[ch]], ssem[b], add=True)

    def wait_scatter(ch, b):
        pltpu.make_async_copy(rows[b], acc.at[dstbuf.at[ch]], ssem[b]).wait()

    for p in range(2):
        q = c * 2 + p

        def zacc(j, _):
            pltpu.sync_copy(zbuf, acc.at[pl.ds(t * ZR + j * ZB, ZB)])
            return 0

        lax.fori_loop(0, ZR // ZB, zacc, 0)
        pltpu.sync_copy(g_hbm.at[q].at[pl.ds(t * GPT, GPT)],
                        gbuf.at[pl.ds(t * GPT, GPT)])
        plsc.subcore_barrier()

        for h in range(CPT // GRP):
            base = t * CPT + h * GRP
            pltpu.sync_copy(src_hbm.at[pl.ds(base, GRP)], srcbuf)
            pltpu.sync_copy(dst_hbm.at[pl.ds(base, GRP)], dstbuf)
            for l in range(NBUF - 1):
                fire_gather(l, l)

            def step(k, _):
                for l in range(NBUF):
                    s = NBUF * k + l
                    wait_gather(s, l)
                    fire_scatter(s, l)
                    tl = (l + NBUF - 1) % NBUF

                    @pl.when(s + NBUF - 1 < GRP)
                    def _():
                        @pl.when(s >= 1)
                        def _():
                            wait_scatter(s - 1, tl)

                        fire_gather(s + NBUF - 1, tl)

                return 0

            lax.fori_loop(0, GRP // NBUF, step, 0)
            for l in range(NBUF):
                wait_scatter(GRP - NBUF + l, l)

        plsc.subcore_barrier()
        pltpu.sync_copy(acc.at[pl.ds(t * ZR, ZR)],
                        s_hbm.at[q].at[pl.ds(t * ZR, ZR)])
        plsc.subcore_barrier()


def _tc1_body(x_ref, w_ref, d_ref, g_ref, dis_ref):
    deg = d_ref[0] + d_ref[1] + 1.0
    dis = lax.rsqrt(deg)
    h = jnp.dot(x_ref[...], w_ref[...], preferred_element_type=jnp.float32)
    g = dis[:, None] * h
    for q in range(NG):
        g_ref[q] = g[:, q * GW:(q + 1) * GW]
    dis_ref[...] = dis


_tc1 = pl.pallas_call(
    _tc1_body,
    grid=(NBLK,),
    in_specs=[
        pl.BlockSpec((BLK, D), lambda i: (i, 0)),
        pl.BlockSpec((D, D), lambda i: (0, 0)),
        pl.BlockSpec((NC, BLK), lambda i: (0, i)),
    ],
    out_specs=[
        pl.BlockSpec((NG, BLK, GW), lambda i: (0, i, 0)),
        pl.BlockSpec((BLK,), lambda i: (i,)),
    ],
    out_shape=[
        jax.ShapeDtypeStruct((NG, N, GW), jnp.float32),
        jax.ShapeDtypeStruct((N,), jnp.float32),
    ],
)


def _tc2_body(s_ref, g_ref, dis_ref, b_ref, o_ref):
    dis = dis_ref[...][:, None]
    for q in range(NG):
        o_ref[:, q * GW:(q + 1) * GW] = (
            dis * (s_ref[q] + g_ref[q]) + b_ref[0, q * GW:(q + 1) * GW])


_tc2 = pl.pallas_call(
    _tc2_body,
    grid=(NBLK,),
    in_specs=[
        pl.BlockSpec((NG, BLK, GW), lambda i: (0, i, 0)),
        pl.BlockSpec((NG, BLK, GW), lambda i: (0, i, 0)),
        pl.BlockSpec((BLK,), lambda i: (i,)),
        pl.BlockSpec((1, D), lambda i: (0, 0)),
    ],
    out_specs=pl.BlockSpec((BLK, D), lambda i: (i, 0)),
    out_shape=jax.ShapeDtypeStruct((N, D), jnp.float32),
)


def kernel(x, W, b, edge_index):
    if edge_index.dtype == jnp.int64:
        ei = jax.lax.bitcast_convert_type(edge_index, jnp.int32)[:, :, 0]
    else:
        ei = edge_index.astype(jnp.int32)
    pad = E_PAD - E
    pad_dst = DUMMY + (jnp.arange(pad, dtype=jnp.int32) % 128)
    src = jnp.concatenate([ei[0], jnp.zeros((pad,), jnp.int32)]).reshape(NCHUNK, CH)
    dst = jnp.concatenate([ei[1], pad_dst]).reshape(NCHUNK, CH)

    d = _deg_kernel(dst)
    g, dis = _tc1(x, W, d)
    s = _edge_kernel(src, dst, g)
    return _tc2(s, g, dis, b.reshape(1, D))
